# baseline (speedup 1.0000x reference)
import functools

import jax
import jax.numpy as jnp
import numpy as np
from jax import lax
from jax.experimental import pallas as pl
from jax.experimental.pallas import tpu as pltpu
from jax.experimental.pallas import tpu_sc as plsc

_B, _N, _C = 32, 8192, 64
_NPOINTS = 2048
_NW = 32
_PER_W = (_B * _NPOINTS) // _NW
_CHUNK = 128
_NCHUNK = _PER_W // _CHUNK
_PAIRS = _CHUNK // 2

_IDX_CONST = None


def _index_consts():
    global _IDX_CONST
    if _IDX_CONST is None:
        with jax.ensure_compile_time_eval():
            idx = jax.random.randint(
                jax.random.key(42), (_B, _NPOINTS), 0, _N, dtype=jnp.int32)
            flat = idx + jnp.arange(_B, dtype=jnp.int32)[:, None] * _N
            flat = np.asarray(flat).reshape(_NW * _NCHUNK, _CHUNK)
        _IDX_CONST = (flat >> 1, (flat & 1) << 6)
    return _IDX_CONST


def _sampler_body(table_hbm, pidx_hbm, hoff_hbm, out_hbm,
                  pidx_v, hoff_v, gbuf, cbuf, gsem0, gsem1, osem0, osem1):
    gsems, osems = (gsem0, gsem1), (osem0, osem1)
    wid = lax.axis_index("s") * 2 + lax.axis_index("c")
    pltpu.sync_copy(pidx_hbm.at[pl.ds(wid * _NCHUNK, _NCHUNK)], pidx_v)
    pltpu.sync_copy(hoff_hbm.at[pl.ds(wid * _NCHUNK, _NCHUNK)], hoff_v)
    out_base = wid * _NCHUNK * _PAIRS

    def gather(j):
        ph = j % 2
        return pltpu.async_copy(
            table_hbm.at[pidx_v.at[j]], gbuf.at[ph], gsems[ph])

    def select(j):
        ph = j % 2
        src = gbuf.at[ph]
        dst = cbuf.at[ph]
        iot = lax.iota(jnp.int32, 16)
        rvecs, hvecs, prows, pcols = [], [], [], []
        for g in range(_CHUNK // 16):
            rv = iot + (16 * g)
            rvecs.append(rv)
            hvecs.append(hoff_v[j, pl.ds(16 * g, 16)])
            prows.append(rv >> 1)
            pcols.append((rv & 1) * _C)

        def body(c, carry):
            for g in range(_CHUNK // 16):
                vals = plsc.load_gather(src, [rvecs[g], hvecs[g] + c])
                plsc.store_scatter(dst, [prows[g], pcols[g] + c], vals)
            return carry

        lax.fori_loop(0, _C, body, 0)

    gh = {0: gather(0)}
    oh = {}
    for j in range(_NCHUNK):
        ph = j % 2
        gh[j].wait()
        if j + 1 < _NCHUNK:
            gh[j + 1] = gather(j + 1)
        if j >= 2:
            oh[j - 2].wait()
        select(j)
        oh[j] = pltpu.async_copy(
            cbuf.at[ph],
            out_hbm.at[pl.ds(out_base + j * _PAIRS, _PAIRS)], osems[ph])
    oh[_NCHUNK - 2].wait()
    oh[_NCHUNK - 1].wait()


@functools.partial(jax.jit, static_argnames=())
def _sampler(table, pidx2d, hoff2d):
    mesh = plsc.VectorSubcoreMesh(core_axis_name="c", subcore_axis_name="s")
    call = pl.kernel(
        _sampler_body,
        out_type=jax.ShapeDtypeStruct((_B * _NPOINTS // 2, 2 * _C),
                                      jnp.float32),
        mesh=mesh,
        scratch_types=[
            pltpu.VMEM((_NCHUNK, _CHUNK), jnp.int32),
            pltpu.VMEM((_NCHUNK, _CHUNK), jnp.int32),
            pltpu.VMEM((2, _CHUNK, 2 * _C), jnp.float32),
            pltpu.VMEM((2, _PAIRS, 2 * _C), jnp.float32),
            pltpu.SemaphoreType.DMA,
            pltpu.SemaphoreType.DMA,
            pltpu.SemaphoreType.DMA,
            pltpu.SemaphoreType.DMA,
        ],
        compiler_params=pltpu.CompilerParams(needs_layout_passes=False),
    )
    return call(table, pidx2d, hoff2d)


def kernel(inputs):
    table = inputs.reshape(_B * _N // 2, 2 * _C)
    pidx, hoff = _index_consts()
    out = _sampler(table, jnp.asarray(pidx), jnp.asarray(hoff))
    return out.reshape(_B, _NPOINTS, _C)


# speedup vs baseline: 4.9304x; 4.9304x over previous
import functools

import jax
import jax.numpy as jnp
import numpy as np
from jax import lax
from jax.experimental import pallas as pl
from jax.experimental.pallas import tpu as pltpu
from jax.experimental.pallas import tpu_sc as plsc

_B, _N, _C = 32, 8192, 64
_NPOINTS = 2048
_SROWS = 4
_NSTRIPE = _C // _SROWS

_IDX_CONST = None


def _index_consts() -> np.ndarray:
    global _IDX_CONST
    if _IDX_CONST is None:
        with jax.ensure_compile_time_eval():
            idx = jax.random.randint(
                jax.random.key(42), (_B, _NPOINTS), 0, _N, dtype=jnp.int32)
            _IDX_CONST = np.asarray(idx)
    return _IDX_CONST


def _sampler_body(table_hbm, idx_hbm, out_hbm,
                  idx_v, inbuf, outbuf, isem0, isem1, osem0, osem1):
    isems, osems = (isem0, isem1), (osem0, osem1)
    b = lax.axis_index("s") * 2 + lax.axis_index("c")
    row0 = b * _C
    pltpu.sync_copy(idx_hbm.at[b], idx_v)
    rsplats = [jnp.full((16,), r, jnp.int32) for r in range(_SROWS)]

    def start_in(s):
        ph = s % 2
        return pltpu.async_copy(
            table_hbm.at[pl.ds(row0 + s * _SROWS, _SROWS)],
            inbuf.at[ph], isems[ph])

    def gather(s):
        ph = s % 2
        src = inbuf.at[ph]
        dst = outbuf.at[ph]

        def body(i, carry):
            for u in range(2):
                col = (2 * i + u) * 16
                idxv = idx_v[pl.ds(col, 16)]
                for r in range(_SROWS):
                    v = plsc.load_gather(src, [rsplats[r], idxv])
                    dst[r, pl.ds(col, 16)] = v
            return carry

        lax.fori_loop(0, _NPOINTS // 32, body, 0)

    ih = {0: start_in(0)}
    oh = {}
    for s in range(_NSTRIPE):
        ph = s % 2
        ih[s].wait()
        if s + 1 < _NSTRIPE:
            ih[s + 1] = start_in(s + 1)
        if s >= 2:
            oh[s - 2].wait()
        gather(s)
        oh[s] = pltpu.async_copy(
            outbuf.at[ph],
            out_hbm.at[pl.ds(row0 + s * _SROWS, _SROWS)], osems[ph])
    oh[_NSTRIPE - 2].wait()
    oh[_NSTRIPE - 1].wait()


@functools.partial(jax.jit, static_argnames=())
def _sampler(table, idx):
    mesh = plsc.VectorSubcoreMesh(core_axis_name="c", subcore_axis_name="s")
    call = pl.kernel(
        _sampler_body,
        out_type=jax.ShapeDtypeStruct((_B * _C, _NPOINTS), jnp.float32),
        mesh=mesh,
        scratch_types=[
            pltpu.VMEM((_NPOINTS,), jnp.int32),
            pltpu.VMEM((2, _SROWS, _N), jnp.float32),
            pltpu.VMEM((2, _SROWS, _NPOINTS), jnp.float32),
            pltpu.SemaphoreType.DMA,
            pltpu.SemaphoreType.DMA,
            pltpu.SemaphoreType.DMA,
            pltpu.SemaphoreType.DMA,
        ],
        compiler_params=pltpu.CompilerParams(needs_layout_passes=False),
    )
    return call(table, idx)


def kernel(inputs):
    table = jnp.transpose(inputs, (0, 2, 1)).reshape(_B * _C, _N)
    idx = jnp.asarray(_index_consts())
    out = _sampler(table, idx)
    return jnp.transpose(out.reshape(_B, _C, _NPOINTS), (0, 2, 1))


# speedup vs baseline: 6.6858x; 1.3560x over previous
import functools

import jax
import jax.numpy as jnp
import numpy as np
from jax import lax
from jax.experimental import pallas as pl
from jax.experimental.pallas import tpu as pltpu
from jax.experimental.pallas import tpu_sc as plsc

_B, _N, _C = 32, 8192, 64
_NPOINTS = 2048
_SROWS = 4
_NSTRIPE = _C // _SROWS

_IDX_CONST = None


def _index_consts() -> np.ndarray:
    global _IDX_CONST
    if _IDX_CONST is None:
        with jax.ensure_compile_time_eval():
            idx = jax.random.randint(
                jax.random.key(42), (_B, _NPOINTS), 0, _N, dtype=jnp.int32)
            _IDX_CONST = np.asarray(idx)
    return _IDX_CONST


def _sampler_body(table_hbm, idx_hbm, out_hbm,
                  idx_v, inbuf, outbuf, isem0, isem1, isem2, osem0, osem1):
    isems, osems = (isem0, isem1, isem2), (osem0, osem1)
    b = lax.axis_index("s") * 2 + lax.axis_index("c")
    row0 = b * _C
    pltpu.sync_copy(idx_hbm.at[b], idx_v)
    rsplats = [jnp.full((16,), r, jnp.int32) for r in range(_SROWS)]

    def start_in(s):
        ph = s % 3
        return pltpu.async_copy(
            table_hbm.at[pl.ds(row0 + s * _SROWS, _SROWS)],
            inbuf.at[ph], isems[ph])

    def gather(s):
        ph = s % 3
        src = inbuf.at[ph]
        dst = outbuf.at[s % 2]

        def body(i, carry):
            base = i * 64
            idxvs = [idx_v[pl.ds(base + u * 16, 16)] for u in range(4)]
            vals = [plsc.load_gather(src, [rsplats[r], idxvs[u]])
                    for u in range(4) for r in range(_SROWS)]
            for u in range(4):
                for r in range(_SROWS):
                    dst[r, pl.ds(base + u * 16, 16)] = vals[u * _SROWS + r]
            return carry

        lax.fori_loop(0, _NPOINTS // 64, body, 0)

    ih = {}
    for t in range(3):
        ih[t] = start_in(t)
    oh = {}
    for s in range(_NSTRIPE):
        ih[s].wait()
        if s >= 2:
            oh[s - 2].wait()
        gather(s)
        if s + 3 < _NSTRIPE:
            ih[s + 3] = start_in(s + 3)
        oh[s] = pltpu.async_copy(
            outbuf.at[s % 2],
            out_hbm.at[pl.ds(row0 + s * _SROWS, _SROWS)], osems[s % 2])
    oh[_NSTRIPE - 2].wait()
    oh[_NSTRIPE - 1].wait()


@functools.partial(jax.jit, static_argnames=())
def _sampler(table, idx):
    mesh = plsc.VectorSubcoreMesh(core_axis_name="c", subcore_axis_name="s")
    call = pl.kernel(
        _sampler_body,
        out_type=jax.ShapeDtypeStruct((_B * _C, _NPOINTS), jnp.float32),
        mesh=mesh,
        scratch_types=[
            pltpu.VMEM((_NPOINTS,), jnp.int32),
            pltpu.VMEM((3, _SROWS, _N), jnp.float32),
            pltpu.VMEM((2, _SROWS, _NPOINTS), jnp.float32),
            pltpu.SemaphoreType.DMA,
            pltpu.SemaphoreType.DMA,
            pltpu.SemaphoreType.DMA,
            pltpu.SemaphoreType.DMA,
            pltpu.SemaphoreType.DMA,
        ],
        compiler_params=pltpu.CompilerParams(needs_layout_passes=False),
    )
    return call(table, idx)


def kernel(inputs):
    table = jnp.transpose(inputs, (0, 2, 1)).reshape(_B * _C, _N)
    idx = jnp.asarray(_index_consts())
    out = _sampler(table, idx)
    return jnp.transpose(out.reshape(_B, _C, _NPOINTS), (0, 2, 1))


# speedup vs baseline: 6.7592x; 1.0110x over previous
import functools

import jax
import jax.numpy as jnp
import numpy as np
from jax import lax
from jax.experimental import pallas as pl
from jax.experimental.pallas import tpu as pltpu
from jax.experimental.pallas import tpu_sc as plsc

_B, _N, _C = 32, 8192, 64
_NPOINTS = 2048
_SROWS = 4
_NSTRIPE = _C // _SROWS

_IDX_CONST = None


def _index_consts() -> np.ndarray:
    global _IDX_CONST
    if _IDX_CONST is None:
        with jax.ensure_compile_time_eval():
            idx = jax.random.randint(
                jax.random.key(42), (_B, _NPOINTS), 0, _N, dtype=jnp.int32)
            _IDX_CONST = np.asarray(idx)
    return _IDX_CONST


def _sampler_body(table_hbm, idx_hbm, out_hbm,
                  idx_v, inbuf, outbuf, isem0, isem1, isem2, osem0, osem1):
    isems, osems = (isem0, isem1, isem2), (osem0, osem1)
    b = lax.axis_index("s") * 2 + lax.axis_index("c")
    row0 = b * _C
    pltpu.sync_copy(idx_hbm.at[b], idx_v)
    rsplats = [jnp.full((16,), r, jnp.int32) for r in range(_SROWS)]

    def start_in(s):
        ph = s % 3
        return pltpu.async_copy(
            table_hbm.at[pl.ds(row0 + s * _SROWS, _SROWS)],
            inbuf.at[ph], isems[ph])

    def gather(s):
        ph = s % 3
        src = inbuf.at[ph]
        dst = outbuf.at[s % 2]

        def body(i, carry):
            base = i * 128
            idxvs = [idx_v[pl.ds(base + u * 16, 16)] for u in range(8)]
            vals = [plsc.load_gather(src, [rsplats[r], idxvs[u]])
                    for u in range(8) for r in range(_SROWS)]
            for u in range(8):
                for r in range(_SROWS):
                    dst[r, pl.ds(base + u * 16, 16)] = vals[u * _SROWS + r]
            return carry

        lax.fori_loop(0, _NPOINTS // 128, body, 0)

    ih = {}
    for t in range(3):
        ih[t] = start_in(t)
    oh = {}
    for s in range(_NSTRIPE):
        ih[s].wait()
        if s >= 2:
            oh[s - 2].wait()
        gather(s)
        if s + 3 < _NSTRIPE:
            ih[s + 3] = start_in(s + 3)
        oh[s] = pltpu.async_copy(
            outbuf.at[s % 2],
            out_hbm.at[pl.ds(row0 + s * _SROWS, _SROWS)], osems[s % 2])
    oh[_NSTRIPE - 2].wait()
    oh[_NSTRIPE - 1].wait()


@functools.partial(jax.jit, static_argnames=())
def _sampler(table, idx):
    mesh = plsc.VectorSubcoreMesh(core_axis_name="c", subcore_axis_name="s")
    call = pl.kernel(
        _sampler_body,
        out_type=jax.ShapeDtypeStruct((_B * _C, _NPOINTS), jnp.float32),
        mesh=mesh,
        scratch_types=[
            pltpu.VMEM((_NPOINTS,), jnp.int32),
            pltpu.VMEM((3, _SROWS, _N), jnp.float32),
            pltpu.VMEM((2, _SROWS, _NPOINTS), jnp.float32),
            pltpu.SemaphoreType.DMA,
            pltpu.SemaphoreType.DMA,
            pltpu.SemaphoreType.DMA,
            pltpu.SemaphoreType.DMA,
            pltpu.SemaphoreType.DMA,
        ],
        compiler_params=pltpu.CompilerParams(needs_layout_passes=False),
    )
    return call(table, idx)


def kernel(inputs):
    table = jnp.transpose(inputs, (0, 2, 1)).reshape(_B * _C, _N)
    idx = jnp.asarray(_index_consts())
    out = _sampler(table, idx)
    return jnp.transpose(out.reshape(_B, _C, _NPOINTS), (0, 2, 1))
